# baseline (speedup 1.0000x reference)
import functools

import jax
import jax.numpy as jnp
from jax import lax
from jax.experimental import pallas as pl
from jax.experimental.pallas import tpu as pltpu
from jax.experimental.pallas import tpu_sc as plsc

_EPS = 1e-5
_B = 16384
_EMB = 64

_NC = 2
_NS = 16
_NW = _NC * _NS
_BPW = _B // _NW

def _gather_body(uid_hbm, iid_hbm, umf_hbm, imf_hbm, umlp_hbm, imlp_hbm,
                 ug_out, ig_out, um_out, im_out,
                 idx_u, idx_i, buf_a, buf_b, sem_a, sem_b):
    wid = lax.axis_index("s") * _NC + lax.axis_index("c")
    base = wid * _BPW
    pltpu.sync_copy(uid_hbm.at[pl.ds(base, _BPW)], idx_u)
    pltpu.sync_copy(iid_hbm.at[pl.ds(base, _BPW)], idx_i)
    cp_a = pltpu.async_copy(umf_hbm.at[idx_u], buf_a, sem_a)
    cp_b = pltpu.async_copy(imf_hbm.at[idx_i], buf_b, sem_b)
    cp_a.wait()
    pltpu.sync_copy(buf_a, ug_out.at[pl.ds(base, _BPW)])
    cp_a2 = pltpu.async_copy(umlp_hbm.at[idx_u], buf_a, sem_a)
    cp_b.wait()
    pltpu.sync_copy(buf_b, ig_out.at[pl.ds(base, _BPW)])
    cp_b2 = pltpu.async_copy(imlp_hbm.at[idx_i], buf_b, sem_b)
    cp_a2.wait()
    pltpu.sync_copy(buf_a, um_out.at[pl.ds(base, _BPW)])
    cp_b2.wait()
    pltpu.sync_copy(buf_b, im_out.at[pl.ds(base, _BPW)])


@functools.cache
def _make_gather():
    mesh = plsc.VectorSubcoreMesh(
        core_axis_name="c", subcore_axis_name="s",
        num_cores=_NC, num_subcores=_NS)
    return pl.kernel(
        _gather_body,
        out_type=[jax.ShapeDtypeStruct((_B, _EMB), jnp.float32)] * 4,
        mesh=mesh,
        scratch_types=[
            pltpu.VMEM((_BPW,), jnp.int32),
            pltpu.VMEM((_BPW,), jnp.int32),
            pltpu.VMEM((_BPW, _EMB), jnp.float32),
            pltpu.VMEM((_BPW, _EMB), jnp.float32),
            pltpu.SemaphoreType.DMA,
            pltpu.SemaphoreType.DMA,
        ],
        compiler_params=pltpu.CompilerParams(use_tc_tiling_on_sc=False),
    )

_BB = 2048


def _mlp_body(ug, ig, um, im, w1, b1, g1, be1, w2, b2, g2, be2, wf, bf, out):
    inv_std = 1.0 / jnp.sqrt(1.0 + _EPS)
    x = jnp.concatenate([um[...], im[...]], axis=1)
    h = lax.dot_general(x, w1[...], (((1,), (1,)), ((), ())),
                        preferred_element_type=jnp.float32)
    h = (h + b1[...]) * inv_std * g1[...] + be1[...]
    h = jnp.maximum(h, 0.0)
    h = lax.dot_general(h, w2[...], (((1,), (1,)), ((), ())),
                        preferred_element_type=jnp.float32)
    h = (h + b2[...]) * inv_std * g2[...] + be2[...]
    h = jnp.maximum(h, 0.0)
    gmf = ug[...] * ig[...]
    c = jnp.concatenate([gmf, h], axis=1)
    logit = jnp.sum(c * wf[...], axis=1, keepdims=True) + bf[0, 0]
    out[...] = 1.0 / (1.0 + jnp.exp(-logit))


def _make_mlp(interpret=False):
    def whole(shape):
        return pl.BlockSpec(shape, lambda i: (0, 0))

    return pl.pallas_call(
        _mlp_body,
        grid=(_B // _BB,),
        in_specs=[
            pl.BlockSpec((_BB, _EMB), lambda i: (i, 0)),
            pl.BlockSpec((_BB, _EMB), lambda i: (i, 0)),
            pl.BlockSpec((_BB, _EMB), lambda i: (i, 0)),
            pl.BlockSpec((_BB, _EMB), lambda i: (i, 0)),
            whole((128, 128)),
            whole((1, 128)),
            whole((1, 128)),
            whole((1, 128)),
            whole((64, 128)),
            whole((1, 64)),
            whole((1, 64)),
            whole((1, 64)),
            whole((1, 128)),
            whole((1, 1)),
        ],
        out_specs=pl.BlockSpec((_BB, 1), lambda i: (i, 0)),
        out_shape=jax.ShapeDtypeStruct((_B, 1), jnp.float32),
        compiler_params=pltpu.CompilerParams(
            dimension_semantics=("arbitrary",),
        ),
        interpret=interpret,
    )


_mlp = _make_mlp()


def kernel(user_ids, item_ids, U_mf, I_mf, U_mlp, I_mlp,
           W1, b1, g1, be1, W2, b2, g2, be2, Wf, bf):
    uid = user_ids.astype(jnp.int32)
    iid = item_ids.astype(jnp.int32)
    ug, ig, um, im = _make_gather()(uid, iid, U_mf, I_mf, U_mlp, I_mlp)
    return _mlp(
        ug, ig, um, im,
        W1, b1.reshape(1, 128), g1.reshape(1, 128), be1.reshape(1, 128),
        W2, b2.reshape(1, 64), g2.reshape(1, 64), be2.reshape(1, 64),
        Wf, bf.reshape(1, 1),
    )


# speedup vs baseline: 1.3795x; 1.3795x over previous
import functools

import jax
import jax.numpy as jnp
from jax import lax
from jax.experimental import pallas as pl
from jax.experimental.pallas import tpu as pltpu
from jax.experimental.pallas import tpu_sc as plsc

_EPS = 1e-5
_B = 16384
_N = 1000000
_EMB = 64

_NC = 2
_NS = 16
_NW = _NC * _NS
_BPW = _B // _NW

_W = 2048
_RGRID = (_N + _W - 1) // _W


def _repack_body(a_ref, b_ref, out_ref):
    a = a_ref[...].astype(jnp.bfloat16)
    b = b_ref[...].astype(jnp.bfloat16)
    at = jnp.transpose(a, (1, 0))
    bt = jnp.transpose(b, (1, 0))
    c = jnp.concatenate([at, bt], axis=1)
    cp = c.reshape(_W // 2, 2, 128)
    lo = lax.bitcast_convert_type(cp[:, 0, :], jnp.uint16)
    hi = lax.bitcast_convert_type(cp[:, 1, :], jnp.uint16)
    out_ref[...] = lo.astype(jnp.uint32) | (hi.astype(jnp.uint32) << 16)


@functools.cache
def _make_repack(interpret=False):
    return pl.pallas_call(
        _repack_body,
        grid=(_RGRID,),
        in_specs=[
            pl.BlockSpec((_EMB, _W), lambda i: (0, i)),
            pl.BlockSpec((_EMB, _W), lambda i: (0, i)),
        ],
        out_specs=pl.BlockSpec((_W // 2, 128), lambda i: (i, 0)),
        out_shape=jax.ShapeDtypeStruct((_N // 2, 128), jnp.uint32),
        compiler_params=pltpu.CompilerParams(
            dimension_semantics=("arbitrary",),
        ),
        interpret=interpret,
    )


def _gather_body(idx_hbm, tbl_hbm, out, idx_v, buf_a, buf_b, sem_a, sem_b):
    wid = lax.axis_index("s") * _NC + lax.axis_index("c")
    base = wid * _BPW
    half = _BPW // 2
    pltpu.sync_copy(idx_hbm.at[pl.ds(base, _BPW)], idx_v)
    cp_a = pltpu.async_copy(tbl_hbm.at[idx_v.at[pl.ds(0, half)]], buf_a, sem_a)
    cp_b = pltpu.async_copy(tbl_hbm.at[idx_v.at[pl.ds(half, half)]], buf_b,
                            sem_b)
    cp_a.wait()
    pltpu.sync_copy(buf_a, out.at[pl.ds(base, half)])
    cp_b.wait()
    pltpu.sync_copy(buf_b, out.at[pl.ds(base + half, half)])


@functools.cache
def _make_gather():
    mesh = plsc.VectorSubcoreMesh(
        core_axis_name="c", subcore_axis_name="s",
        num_cores=_NC, num_subcores=_NS)
    half = _BPW // 2
    return pl.kernel(
        _gather_body,
        out_type=jax.ShapeDtypeStruct((_B, 128), jnp.uint32),
        mesh=mesh,
        scratch_types=[
            pltpu.VMEM((_BPW,), jnp.int32),
            pltpu.VMEM((half, 128), jnp.uint32),
            pltpu.VMEM((half, 128), jnp.uint32),
            pltpu.SemaphoreType.DMA,
            pltpu.SemaphoreType.DMA,
        ],
        compiler_params=pltpu.CompilerParams(use_tc_tiling_on_sc=True),
    )


_BB = 2048


def _unpack_rows(packed, parity_col):
    lo = lax.bitcast_convert_type(
        (packed & jnp.uint32(0xFFFF)).astype(jnp.uint16), jnp.bfloat16)
    hi = lax.bitcast_convert_type(
        (packed >> 16).astype(jnp.uint16), jnp.bfloat16)
    return jnp.where(parity_col == 1, hi, lo).astype(jnp.float32)


def _mlp_body(up_ref, ip_ref, upar_ref, ipar_ref,
              w1, b1, g1, be1, w2, b2, g2, be2, wf, bf, out):
    inv_std = 1.0 / jnp.sqrt(1.0 + _EPS)
    u = _unpack_rows(up_ref[...], upar_ref[...])
    i = _unpack_rows(ip_ref[...], ipar_ref[...])
    x = jnp.concatenate([u[:, _EMB:], i[:, _EMB:]], axis=1)
    h = lax.dot_general(x, w1[...], (((1,), (1,)), ((), ())),
                        preferred_element_type=jnp.float32)
    h = (h + b1[...]) * inv_std * g1[...] + be1[...]
    h = jnp.maximum(h, 0.0)
    h = lax.dot_general(h, w2[...], (((1,), (1,)), ((), ())),
                        preferred_element_type=jnp.float32)
    h = (h + b2[...]) * inv_std * g2[...] + be2[...]
    h = jnp.maximum(h, 0.0)
    gmf = u[:, :_EMB] * i[:, :_EMB]
    c = jnp.concatenate([gmf, h], axis=1)
    logit = jnp.sum(c * wf[...], axis=1, keepdims=True) + bf[0, 0]
    out[...] = 1.0 / (1.0 + jnp.exp(-logit))


@functools.cache
def _make_mlp(interpret=False):
    def whole(shape):
        return pl.BlockSpec(shape, lambda i: (0, 0))

    return pl.pallas_call(
        _mlp_body,
        grid=(_B // _BB,),
        in_specs=[
            pl.BlockSpec((_BB, 128), lambda i: (i, 0)),
            pl.BlockSpec((_BB, 128), lambda i: (i, 0)),
            pl.BlockSpec((_BB, 1), lambda i: (i, 0)),
            pl.BlockSpec((_BB, 1), lambda i: (i, 0)),
            whole((128, 128)),
            whole((1, 128)),
            whole((1, 128)),
            whole((1, 128)),
            whole((64, 128)),
            whole((1, 64)),
            whole((1, 64)),
            whole((1, 64)),
            whole((1, 128)),
            whole((1, 1)),
        ],
        out_specs=pl.BlockSpec((_BB, 1), lambda i: (i, 0)),
        out_shape=jax.ShapeDtypeStruct((_B, 1), jnp.float32),
        compiler_params=pltpu.CompilerParams(
            dimension_semantics=("arbitrary",),
        ),
        interpret=interpret,
    )


def kernel(user_ids, item_ids, U_mf, I_mf, U_mlp, I_mlp,
           W1, b1, g1, be1, W2, b2, g2, be2, Wf, bf):
    uid = user_ids.astype(jnp.int32)
    iid = item_ids.astype(jnp.int32)
    repack = _make_repack()
    ut = repack(U_mf.T, U_mlp.T)
    it = repack(I_mf.T, I_mlp.T)
    g = _make_gather()
    urows = g(uid // 2, ut)
    irows = g(iid // 2, it)
    return _make_mlp()(
        urows, irows,
        (uid % 2).astype(jnp.uint32).reshape(_B, 1),
        (iid % 2).astype(jnp.uint32).reshape(_B, 1),
        W1, b1.reshape(1, 128), g1.reshape(1, 128), be1.reshape(1, 128),
        W2, b2.reshape(1, 64), g2.reshape(1, 64), be2.reshape(1, 64),
        Wf, bf.reshape(1, 1),
    )


# speedup vs baseline: 1.5951x; 1.1563x over previous
import functools

import jax
import jax.numpy as jnp
from jax import lax
from jax.experimental import pallas as pl
from jax.experimental.pallas import tpu as pltpu
from jax.experimental.pallas import tpu_sc as plsc

_EPS = 1e-5
_B = 16384
_N = 1000000
_EMB = 64

_NC = 2
_NS = 16
_NW = _NC * _NS
_BPW = _B // _NW

_W = 2048
_RGRID = (_N + _W - 1) // _W


def _repack_body(a_ref, b_ref, out_ref):
    at = jnp.transpose(a_ref[...], (1, 0))
    bt = jnp.transpose(b_ref[...], (1, 0))
    out_ref[...] = jnp.concatenate([at, bt], axis=1)


@functools.cache
def _make_repack(interpret=False):
    return pl.pallas_call(
        _repack_body,
        grid=(_RGRID,),
        in_specs=[
            pl.BlockSpec((_EMB, _W), lambda i: (0, i)),
            pl.BlockSpec((_EMB, _W), lambda i: (0, i)),
        ],
        out_specs=pl.BlockSpec((_W, 128), lambda i: (i, 0)),
        out_shape=jax.ShapeDtypeStruct((_N, 128), jnp.float32),
        compiler_params=pltpu.CompilerParams(
            dimension_semantics=("arbitrary",),
        ),
        interpret=interpret,
    )


def _gather_body(idx_hbm, tbl_hbm, out, idx_v, buf_a, buf_b, sem_a, sem_b):
    wid = lax.axis_index("s") * _NC + lax.axis_index("c")
    base = wid * _BPW
    half = _BPW // 2
    pltpu.sync_copy(idx_hbm.at[pl.ds(base, _BPW)], idx_v)
    cp_a = pltpu.async_copy(tbl_hbm.at[idx_v.at[pl.ds(0, half)]], buf_a, sem_a)
    cp_b = pltpu.async_copy(tbl_hbm.at[idx_v.at[pl.ds(half, half)]], buf_b,
                            sem_b)
    cp_a.wait()
    pltpu.sync_copy(buf_a, out.at[pl.ds(base, half)])
    cp_b.wait()
    pltpu.sync_copy(buf_b, out.at[pl.ds(base + half, half)])


@functools.cache
def _make_gather():
    mesh = plsc.VectorSubcoreMesh(
        core_axis_name="c", subcore_axis_name="s",
        num_cores=_NC, num_subcores=_NS)
    half = _BPW // 2
    return pl.kernel(
        _gather_body,
        out_type=jax.ShapeDtypeStruct((_B, 128), jnp.float32),
        mesh=mesh,
        scratch_types=[
            pltpu.VMEM((_BPW,), jnp.int32),
            pltpu.VMEM((half, 128), jnp.float32),
            pltpu.VMEM((half, 128), jnp.float32),
            pltpu.SemaphoreType.DMA,
            pltpu.SemaphoreType.DMA,
        ],
        compiler_params=pltpu.CompilerParams(use_tc_tiling_on_sc=True),
    )


_BB = 2048


def _mlp_body(u_ref, i_ref, w1, b1, g1, be1, w2, b2, g2, be2, wf, bf, out):
    inv_std = 1.0 / jnp.sqrt(1.0 + _EPS)
    u = u_ref[...]
    i = i_ref[...]
    x = jnp.concatenate([u[:, _EMB:], i[:, _EMB:]], axis=1)
    h = lax.dot_general(x, w1[...], (((1,), (1,)), ((), ())),
                        preferred_element_type=jnp.float32)
    h = (h + b1[...]) * inv_std * g1[...] + be1[...]
    h = jnp.maximum(h, 0.0)
    h = lax.dot_general(h, w2[...], (((1,), (1,)), ((), ())),
                        preferred_element_type=jnp.float32)
    h = (h + b2[...]) * inv_std * g2[...] + be2[...]
    h = jnp.maximum(h, 0.0)
    gmf = u[:, :_EMB] * i[:, :_EMB]
    c = jnp.concatenate([gmf, h], axis=1)
    logit = jnp.sum(c * wf[...], axis=1, keepdims=True) + bf[0, 0]
    out[...] = 1.0 / (1.0 + jnp.exp(-logit))


@functools.cache
def _make_mlp(interpret=False):
    def whole(shape):
        return pl.BlockSpec(shape, lambda i: (0, 0))

    return pl.pallas_call(
        _mlp_body,
        grid=(_B // _BB,),
        in_specs=[
            pl.BlockSpec((_BB, 128), lambda i: (i, 0)),
            pl.BlockSpec((_BB, 128), lambda i: (i, 0)),
            whole((128, 128)),
            whole((1, 128)),
            whole((1, 128)),
            whole((1, 128)),
            whole((64, 128)),
            whole((1, 64)),
            whole((1, 64)),
            whole((1, 64)),
            whole((1, 128)),
            whole((1, 1)),
        ],
        out_specs=pl.BlockSpec((_BB, 1), lambda i: (i, 0)),
        out_shape=jax.ShapeDtypeStruct((_B, 1), jnp.float32),
        compiler_params=pltpu.CompilerParams(
            dimension_semantics=("arbitrary",),
        ),
        interpret=interpret,
    )


def kernel(user_ids, item_ids, U_mf, I_mf, U_mlp, I_mlp,
           W1, b1, g1, be1, W2, b2, g2, be2, Wf, bf):
    uid = user_ids.astype(jnp.int32)
    iid = item_ids.astype(jnp.int32)
    repack = _make_repack()
    ut = repack(U_mf.T, U_mlp.T)
    it = repack(I_mf.T, I_mlp.T)
    g = _make_gather()
    urows = g(uid, ut)
    irows = g(iid, it)
    return _make_mlp()(
        urows, irows,
        W1, b1.reshape(1, 128), g1.reshape(1, 128), be1.reshape(1, 128),
        W2, b2.reshape(1, 64), g2.reshape(1, 64), be2.reshape(1, 64),
        Wf, bf.reshape(1, 1),
    )


# speedup vs baseline: 1.5955x; 1.0003x over previous
import functools

import jax
import jax.numpy as jnp
from jax import lax
from jax.experimental import pallas as pl
from jax.experimental.pallas import tpu as pltpu
from jax.experimental.pallas import tpu_sc as plsc

_EPS = 1e-5
_B = 16384
_N = 1000000
_EMB = 64

_NC = 2
_NS = 16
_NW = _NC * _NS
_BPW = _B // _NW

_W = 2048
_RGRID = (_N + _W - 1) // _W


def _repack_body(a_ref, b_ref, out_ref):
    eye = jnp.eye(_EMB, dtype=jnp.float32)
    at = lax.dot_general(a_ref[...], eye, (((0,), (0,)), ((), ())),
                         preferred_element_type=jnp.float32)
    bt = lax.dot_general(b_ref[...], eye, (((0,), (0,)), ((), ())),
                         preferred_element_type=jnp.float32)
    out_ref[...] = jnp.concatenate([at, bt], axis=1)


@functools.cache
def _make_repack(interpret=False):
    return pl.pallas_call(
        _repack_body,
        grid=(_RGRID,),
        in_specs=[
            pl.BlockSpec((_EMB, _W), lambda i: (0, i)),
            pl.BlockSpec((_EMB, _W), lambda i: (0, i)),
        ],
        out_specs=pl.BlockSpec((_W, 128), lambda i: (i, 0)),
        out_shape=jax.ShapeDtypeStruct((_N, 128), jnp.float32),
        compiler_params=pltpu.CompilerParams(
            dimension_semantics=("arbitrary",),
        ),
        interpret=interpret,
    )


def _gather_body(idx_hbm, tbl_hbm, out, idx_v, buf_a, buf_b, sem_a, sem_b):
    wid = lax.axis_index("s") * _NC + lax.axis_index("c")
    base = wid * _BPW
    half = _BPW // 2
    pltpu.sync_copy(idx_hbm.at[pl.ds(base, _BPW)], idx_v)
    cp_a = pltpu.async_copy(tbl_hbm.at[idx_v.at[pl.ds(0, half)]], buf_a, sem_a)
    cp_b = pltpu.async_copy(tbl_hbm.at[idx_v.at[pl.ds(half, half)]], buf_b,
                            sem_b)
    cp_a.wait()
    pltpu.sync_copy(buf_a, out.at[pl.ds(base, half)])
    cp_b.wait()
    pltpu.sync_copy(buf_b, out.at[pl.ds(base + half, half)])


@functools.cache
def _make_gather():
    mesh = plsc.VectorSubcoreMesh(
        core_axis_name="c", subcore_axis_name="s",
        num_cores=_NC, num_subcores=_NS)
    half = _BPW // 2
    return pl.kernel(
        _gather_body,
        out_type=jax.ShapeDtypeStruct((_B, 128), jnp.float32),
        mesh=mesh,
        scratch_types=[
            pltpu.VMEM((_BPW,), jnp.int32),
            pltpu.VMEM((half, 128), jnp.float32),
            pltpu.VMEM((half, 128), jnp.float32),
            pltpu.SemaphoreType.DMA,
            pltpu.SemaphoreType.DMA,
        ],
        compiler_params=pltpu.CompilerParams(use_tc_tiling_on_sc=True),
    )


_BB = 2048


def _mlp_body(u_ref, i_ref, w1, b1, g1, be1, w2, b2, g2, be2, wf, bf, out):
    inv_std = 1.0 / jnp.sqrt(1.0 + _EPS)
    u = u_ref[...]
    i = i_ref[...]
    x = jnp.concatenate([u[:, _EMB:], i[:, _EMB:]], axis=1)
    h = lax.dot_general(x, w1[...], (((1,), (1,)), ((), ())),
                        preferred_element_type=jnp.float32)
    h = (h + b1[...]) * inv_std * g1[...] + be1[...]
    h = jnp.maximum(h, 0.0)
    h = lax.dot_general(h, w2[...], (((1,), (1,)), ((), ())),
                        preferred_element_type=jnp.float32)
    h = (h + b2[...]) * inv_std * g2[...] + be2[...]
    h = jnp.maximum(h, 0.0)
    gmf = u[:, :_EMB] * i[:, :_EMB]
    c = jnp.concatenate([gmf, h], axis=1)
    logit = jnp.sum(c * wf[...], axis=1, keepdims=True) + bf[0, 0]
    out[...] = 1.0 / (1.0 + jnp.exp(-logit))


@functools.cache
def _make_mlp(interpret=False):
    def whole(shape):
        return pl.BlockSpec(shape, lambda i: (0, 0))

    return pl.pallas_call(
        _mlp_body,
        grid=(_B // _BB,),
        in_specs=[
            pl.BlockSpec((_BB, 128), lambda i: (i, 0)),
            pl.BlockSpec((_BB, 128), lambda i: (i, 0)),
            whole((128, 128)),
            whole((1, 128)),
            whole((1, 128)),
            whole((1, 128)),
            whole((64, 128)),
            whole((1, 64)),
            whole((1, 64)),
            whole((1, 64)),
            whole((1, 128)),
            whole((1, 1)),
        ],
        out_specs=pl.BlockSpec((_BB, 1), lambda i: (i, 0)),
        out_shape=jax.ShapeDtypeStruct((_B, 1), jnp.float32),
        compiler_params=pltpu.CompilerParams(
            dimension_semantics=("arbitrary",),
        ),
        interpret=interpret,
    )


def kernel(user_ids, item_ids, U_mf, I_mf, U_mlp, I_mlp,
           W1, b1, g1, be1, W2, b2, g2, be2, Wf, bf):
    uid = user_ids.astype(jnp.int32)
    iid = item_ids.astype(jnp.int32)
    repack = _make_repack()
    ut = repack(U_mf.T, U_mlp.T)
    it = repack(I_mf.T, I_mlp.T)
    g = _make_gather()
    urows = g(uid, ut)
    irows = g(iid, it)
    return _make_mlp()(
        urows, irows,
        W1, b1.reshape(1, 128), g1.reshape(1, 128), be1.reshape(1, 128),
        W2, b2.reshape(1, 64), g2.reshape(1, 64), be2.reshape(1, 64),
        Wf, bf.reshape(1, 1),
    )


# speedup vs baseline: 2.0005x; 1.2538x over previous
import functools

import jax
import jax.numpy as jnp
from jax import lax
from jax.experimental import pallas as pl
from jax.experimental.pallas import tpu as pltpu
from jax.experimental.pallas import tpu_sc as plsc

_EPS = 1e-5
_B = 16384
_N = 1000000
_EMB = 64

_NC = 2
_NS = 16
_NW = _NC * _NS
_BPW = _B // _NW

_W = 4096
_RGRID = (_N + _W - 1) // _W


def _repack_body(a_ref, b_ref, out_ref):
    eye = jnp.eye(_EMB, dtype=jnp.float32)
    at = lax.dot_general(a_ref[...], eye, (((0,), (0,)), ((), ())),
                         preferred_element_type=jnp.float32)
    bt = lax.dot_general(b_ref[...], eye, (((0,), (0,)), ((), ())),
                         preferred_element_type=jnp.float32)
    out_ref[...] = jnp.concatenate([at, bt], axis=1)


@functools.cache
def _make_repack(interpret=False):
    return pl.pallas_call(
        _repack_body,
        grid=(_RGRID,),
        in_specs=[
            pl.BlockSpec((_EMB, _W), lambda i: (0, i)),
            pl.BlockSpec((_EMB, _W), lambda i: (0, i)),
        ],
        out_specs=pl.BlockSpec((_W, 128), lambda i: (i, 0)),
        out_shape=jax.ShapeDtypeStruct((_N, 128), jnp.float32),
        compiler_params=pltpu.CompilerParams(
            dimension_semantics=("parallel",),
        ),
        interpret=interpret,
    )


def _gather_body(idx_hbm, tbl_hbm, out, idx_v, buf_a, buf_b, sem_a, sem_b):
    wid = lax.axis_index("s") * _NC + lax.axis_index("c")
    base = wid * _BPW
    half = _BPW // 2
    pltpu.sync_copy(idx_hbm.at[pl.ds(base, _BPW)], idx_v)
    cp_a = pltpu.async_copy(tbl_hbm.at[idx_v.at[pl.ds(0, half)]], buf_a, sem_a)
    cp_b = pltpu.async_copy(tbl_hbm.at[idx_v.at[pl.ds(half, half)]], buf_b,
                            sem_b)
    cp_a.wait()
    pltpu.sync_copy(buf_a, out.at[pl.ds(base, half)])
    cp_b.wait()
    pltpu.sync_copy(buf_b, out.at[pl.ds(base + half, half)])


@functools.cache
def _make_gather():
    mesh = plsc.VectorSubcoreMesh(
        core_axis_name="c", subcore_axis_name="s",
        num_cores=_NC, num_subcores=_NS)
    half = _BPW // 2
    return pl.kernel(
        _gather_body,
        out_type=jax.ShapeDtypeStruct((_B, 128), jnp.float32),
        mesh=mesh,
        scratch_types=[
            pltpu.VMEM((_BPW,), jnp.int32),
            pltpu.VMEM((half, 128), jnp.float32),
            pltpu.VMEM((half, 128), jnp.float32),
            pltpu.SemaphoreType.DMA,
            pltpu.SemaphoreType.DMA,
        ],
        compiler_params=pltpu.CompilerParams(use_tc_tiling_on_sc=True),
    )


_BB = 2048


def _mlp_body(u_ref, i_ref, w1, b1, g1, be1, w2, b2, g2, be2, wf, bf, out):
    inv_std = 1.0 / jnp.sqrt(1.0 + _EPS)
    u = u_ref[...]
    i = i_ref[...]
    x = jnp.concatenate([u[:, _EMB:], i[:, _EMB:]], axis=1)
    h = lax.dot_general(x, w1[...], (((1,), (1,)), ((), ())),
                        preferred_element_type=jnp.float32)
    h = (h + b1[...]) * inv_std * g1[...] + be1[...]
    h = jnp.maximum(h, 0.0)
    h = lax.dot_general(h, w2[...], (((1,), (1,)), ((), ())),
                        preferred_element_type=jnp.float32)
    h = (h + b2[...]) * inv_std * g2[...] + be2[...]
    h = jnp.maximum(h, 0.0)
    gmf = u[:, :_EMB] * i[:, :_EMB]
    c = jnp.concatenate([gmf, h], axis=1)
    logit = jnp.sum(c * wf[...], axis=1, keepdims=True) + bf[0, 0]
    out[...] = 1.0 / (1.0 + jnp.exp(-logit))


@functools.cache
def _make_mlp(interpret=False):
    def whole(shape):
        return pl.BlockSpec(shape, lambda i: (0, 0))

    return pl.pallas_call(
        _mlp_body,
        grid=(_B // _BB,),
        in_specs=[
            pl.BlockSpec((_BB, 128), lambda i: (i, 0)),
            pl.BlockSpec((_BB, 128), lambda i: (i, 0)),
            whole((128, 128)),
            whole((1, 128)),
            whole((1, 128)),
            whole((1, 128)),
            whole((64, 128)),
            whole((1, 64)),
            whole((1, 64)),
            whole((1, 64)),
            whole((1, 128)),
            whole((1, 1)),
        ],
        out_specs=pl.BlockSpec((_BB, 1), lambda i: (i, 0)),
        out_shape=jax.ShapeDtypeStruct((_B, 1), jnp.float32),
        compiler_params=pltpu.CompilerParams(
            dimension_semantics=("arbitrary",),
        ),
        interpret=interpret,
    )


def kernel(user_ids, item_ids, U_mf, I_mf, U_mlp, I_mlp,
           W1, b1, g1, be1, W2, b2, g2, be2, Wf, bf):
    uid = user_ids.astype(jnp.int32)
    iid = item_ids.astype(jnp.int32)
    repack = _make_repack()
    ut = repack(U_mf.T, U_mlp.T)
    it = repack(I_mf.T, I_mlp.T)
    g = _make_gather()
    urows = g(uid, ut)
    irows = g(iid, it)
    return _make_mlp()(
        urows, irows,
        W1, b1.reshape(1, 128), g1.reshape(1, 128), be1.reshape(1, 128),
        W2, b2.reshape(1, 64), g2.reshape(1, 64), be2.reshape(1, 64),
        Wf, bf.reshape(1, 1),
    )


# speedup vs baseline: 2.2965x; 1.1479x over previous
import functools

import jax
import jax.numpy as jnp
from jax import lax
from jax.experimental import pallas as pl
from jax.experimental.pallas import tpu as pltpu
from jax.experimental.pallas import tpu_sc as plsc

_EPS = 1e-5
_B = 16384
_N = 1000000
_EMB = 64

_NC = 2
_NS = 16
_NW = _NC * _NS
_BPW = _B // _NW

_W = 8192
_RGRID = (_N + _W - 1) // _W


def _repack_body(a_ref, b_ref, out_ref):
    eye = jnp.eye(_EMB, dtype=jnp.float32)
    at = lax.dot_general(a_ref[...], eye, (((0,), (0,)), ((), ())),
                         preferred_element_type=jnp.float32)
    bt = lax.dot_general(b_ref[...], eye, (((0,), (0,)), ((), ())),
                         preferred_element_type=jnp.float32)
    out_ref[...] = jnp.concatenate([at, bt], axis=1)


@functools.cache
def _make_repack(interpret=False):
    return pl.pallas_call(
        _repack_body,
        grid=(_RGRID,),
        in_specs=[
            pl.BlockSpec((_EMB, _W), lambda i: (0, i)),
            pl.BlockSpec((_EMB, _W), lambda i: (0, i)),
        ],
        out_specs=pl.BlockSpec((_W, 128), lambda i: (i, 0)),
        out_shape=jax.ShapeDtypeStruct((_N, 128), jnp.float32),
        compiler_params=pltpu.CompilerParams(
            dimension_semantics=("parallel",),
        ),
        interpret=interpret,
    )


def _gather_body(idx_hbm, tbl_hbm, out, idx_v, buf_a, buf_b, sem_a, sem_b):
    wid = lax.axis_index("s") * _NC + lax.axis_index("c")
    base = wid * _BPW
    half = _BPW // 2
    pltpu.sync_copy(idx_hbm.at[pl.ds(base, _BPW)], idx_v)
    cp_a = pltpu.async_copy(tbl_hbm.at[idx_v.at[pl.ds(0, half)]], buf_a, sem_a)
    cp_b = pltpu.async_copy(tbl_hbm.at[idx_v.at[pl.ds(half, half)]], buf_b,
                            sem_b)
    cp_a.wait()
    pltpu.sync_copy(buf_a, out.at[pl.ds(base, half)])
    cp_b.wait()
    pltpu.sync_copy(buf_b, out.at[pl.ds(base + half, half)])


@functools.cache
def _make_gather():
    mesh = plsc.VectorSubcoreMesh(
        core_axis_name="c", subcore_axis_name="s",
        num_cores=_NC, num_subcores=_NS)
    half = _BPW // 2
    return pl.kernel(
        _gather_body,
        out_type=jax.ShapeDtypeStruct((_B, 128), jnp.float32),
        mesh=mesh,
        scratch_types=[
            pltpu.VMEM((_BPW,), jnp.int32),
            pltpu.VMEM((half, 128), jnp.float32),
            pltpu.VMEM((half, 128), jnp.float32),
            pltpu.SemaphoreType.DMA,
            pltpu.SemaphoreType.DMA,
        ],
        compiler_params=pltpu.CompilerParams(use_tc_tiling_on_sc=True),
    )


_BB = 2048


def _mlp_body(u_ref, i_ref, w1, b1, g1, be1, w2, b2, g2, be2, wf, bf, out):
    inv_std = 1.0 / jnp.sqrt(1.0 + _EPS)
    u = u_ref[...]
    i = i_ref[...]
    x = jnp.concatenate([u[:, _EMB:], i[:, _EMB:]], axis=1)
    h = lax.dot_general(x, w1[...], (((1,), (1,)), ((), ())),
                        preferred_element_type=jnp.float32)
    h = (h + b1[...]) * inv_std * g1[...] + be1[...]
    h = jnp.maximum(h, 0.0)
    h = lax.dot_general(h, w2[...], (((1,), (1,)), ((), ())),
                        preferred_element_type=jnp.float32)
    h = (h + b2[...]) * inv_std * g2[...] + be2[...]
    h = jnp.maximum(h, 0.0)
    gmf = u[:, :_EMB] * i[:, :_EMB]
    c = jnp.concatenate([gmf, h], axis=1)
    logit = jnp.sum(c * wf[...], axis=1, keepdims=True) + bf[0, 0]
    out[...] = 1.0 / (1.0 + jnp.exp(-logit))


@functools.cache
def _make_mlp(interpret=False):
    def whole(shape):
        return pl.BlockSpec(shape, lambda i: (0, 0))

    return pl.pallas_call(
        _mlp_body,
        grid=(_B // _BB,),
        in_specs=[
            pl.BlockSpec((_BB, 128), lambda i: (i, 0)),
            pl.BlockSpec((_BB, 128), lambda i: (i, 0)),
            whole((128, 128)),
            whole((1, 128)),
            whole((1, 128)),
            whole((1, 128)),
            whole((64, 128)),
            whole((1, 64)),
            whole((1, 64)),
            whole((1, 64)),
            whole((1, 128)),
            whole((1, 1)),
        ],
        out_specs=pl.BlockSpec((_BB, 1), lambda i: (i, 0)),
        out_shape=jax.ShapeDtypeStruct((_B, 1), jnp.float32),
        compiler_params=pltpu.CompilerParams(
            dimension_semantics=("arbitrary",),
        ),
        interpret=interpret,
    )


def kernel(user_ids, item_ids, U_mf, I_mf, U_mlp, I_mlp,
           W1, b1, g1, be1, W2, b2, g2, be2, Wf, bf):
    uid = user_ids.astype(jnp.int32)
    iid = item_ids.astype(jnp.int32)
    repack = _make_repack()
    ut = repack(U_mf.T, U_mlp.T)
    it = repack(I_mf.T, I_mlp.T)
    g = _make_gather()
    urows = g(uid, ut)
    irows = g(iid, it)
    return _make_mlp()(
        urows, irows,
        W1, b1.reshape(1, 128), g1.reshape(1, 128), be1.reshape(1, 128),
        W2, b2.reshape(1, 64), g2.reshape(1, 64), be2.reshape(1, 64),
        Wf, bf.reshape(1, 1),
    )


# speedup vs baseline: 2.4517x; 1.0676x over previous
import functools

import jax
import jax.numpy as jnp
from jax import lax
from jax.experimental import pallas as pl
from jax.experimental.pallas import tpu as pltpu
from jax.experimental.pallas import tpu_sc as plsc

_EPS = 1e-5
_B = 16384
_N = 1000000
_EMB = 64

_NC = 2
_NS = 16
_NW = _NC * _NS
_BPW = _B // _NW

_W = 16384
_RGRID = (_N + _W - 1) // _W


def _repack_body(a_ref, b_ref, out_ref):
    eye = jnp.eye(_EMB, dtype=jnp.float32)
    at = lax.dot_general(a_ref[...], eye, (((0,), (0,)), ((), ())),
                         preferred_element_type=jnp.float32)
    bt = lax.dot_general(b_ref[...], eye, (((0,), (0,)), ((), ())),
                         preferred_element_type=jnp.float32)
    out_ref[...] = jnp.concatenate([at, bt], axis=1)


@functools.cache
def _make_repack(interpret=False):
    return pl.pallas_call(
        _repack_body,
        grid=(_RGRID,),
        in_specs=[
            pl.BlockSpec((_EMB, _W), lambda i: (0, i)),
            pl.BlockSpec((_EMB, _W), lambda i: (0, i)),
        ],
        out_specs=pl.BlockSpec((_W, 128), lambda i: (i, 0)),
        out_shape=jax.ShapeDtypeStruct((_N, 128), jnp.float32),
        compiler_params=pltpu.CompilerParams(
            dimension_semantics=("parallel",),
        ),
        interpret=interpret,
    )


def _gather_body(idx_hbm, tbl_hbm, out, idx_v, buf_a, buf_b, sem_a, sem_b):
    wid = lax.axis_index("s") * _NC + lax.axis_index("c")
    base = wid * _BPW
    half = _BPW // 2
    pltpu.sync_copy(idx_hbm.at[pl.ds(base, _BPW)], idx_v)
    cp_a = pltpu.async_copy(tbl_hbm.at[idx_v.at[pl.ds(0, half)]], buf_a, sem_a)
    cp_b = pltpu.async_copy(tbl_hbm.at[idx_v.at[pl.ds(half, half)]], buf_b,
                            sem_b)
    cp_a.wait()
    pltpu.sync_copy(buf_a, out.at[pl.ds(base, half)])
    cp_b.wait()
    pltpu.sync_copy(buf_b, out.at[pl.ds(base + half, half)])


@functools.cache
def _make_gather():
    mesh = plsc.VectorSubcoreMesh(
        core_axis_name="c", subcore_axis_name="s",
        num_cores=_NC, num_subcores=_NS)
    half = _BPW // 2
    return pl.kernel(
        _gather_body,
        out_type=jax.ShapeDtypeStruct((_B, 128), jnp.float32),
        mesh=mesh,
        scratch_types=[
            pltpu.VMEM((_BPW,), jnp.int32),
            pltpu.VMEM((half, 128), jnp.float32),
            pltpu.VMEM((half, 128), jnp.float32),
            pltpu.SemaphoreType.DMA,
            pltpu.SemaphoreType.DMA,
        ],
        compiler_params=pltpu.CompilerParams(use_tc_tiling_on_sc=True),
    )


_BB = 2048


def _mlp_body(u_ref, i_ref, w1, b1, g1, be1, w2, b2, g2, be2, wf, bf, out):
    inv_std = 1.0 / jnp.sqrt(1.0 + _EPS)
    u = u_ref[...]
    i = i_ref[...]
    x = jnp.concatenate([u[:, _EMB:], i[:, _EMB:]], axis=1)
    h = lax.dot_general(x, w1[...], (((1,), (1,)), ((), ())),
                        preferred_element_type=jnp.float32)
    h = (h + b1[...]) * inv_std * g1[...] + be1[...]
    h = jnp.maximum(h, 0.0)
    h = lax.dot_general(h, w2[...], (((1,), (1,)), ((), ())),
                        preferred_element_type=jnp.float32)
    h = (h + b2[...]) * inv_std * g2[...] + be2[...]
    h = jnp.maximum(h, 0.0)
    gmf = u[:, :_EMB] * i[:, :_EMB]
    c = jnp.concatenate([gmf, h], axis=1)
    logit = jnp.sum(c * wf[...], axis=1, keepdims=True) + bf[0, 0]
    out[...] = 1.0 / (1.0 + jnp.exp(-logit))


@functools.cache
def _make_mlp(interpret=False):
    def whole(shape):
        return pl.BlockSpec(shape, lambda i: (0, 0))

    return pl.pallas_call(
        _mlp_body,
        grid=(_B // _BB,),
        in_specs=[
            pl.BlockSpec((_BB, 128), lambda i: (i, 0)),
            pl.BlockSpec((_BB, 128), lambda i: (i, 0)),
            whole((128, 128)),
            whole((1, 128)),
            whole((1, 128)),
            whole((1, 128)),
            whole((64, 128)),
            whole((1, 64)),
            whole((1, 64)),
            whole((1, 64)),
            whole((1, 128)),
            whole((1, 1)),
        ],
        out_specs=pl.BlockSpec((_BB, 1), lambda i: (i, 0)),
        out_shape=jax.ShapeDtypeStruct((_B, 1), jnp.float32),
        compiler_params=pltpu.CompilerParams(
            dimension_semantics=("arbitrary",),
        ),
        interpret=interpret,
    )


def kernel(user_ids, item_ids, U_mf, I_mf, U_mlp, I_mlp,
           W1, b1, g1, be1, W2, b2, g2, be2, Wf, bf):
    uid = user_ids.astype(jnp.int32)
    iid = item_ids.astype(jnp.int32)
    repack = _make_repack()
    ut = repack(U_mf.T, U_mlp.T)
    it = repack(I_mf.T, I_mlp.T)
    g = _make_gather()
    urows = g(uid, ut)
    irows = g(iid, it)
    return _make_mlp()(
        urows, irows,
        W1, b1.reshape(1, 128), g1.reshape(1, 128), be1.reshape(1, 128),
        W2, b2.reshape(1, 64), g2.reshape(1, 64), be2.reshape(1, 64),
        Wf, bf.reshape(1, 1),
    )


# speedup vs baseline: 2.7291x; 1.1132x over previous
import functools

import jax
import jax.numpy as jnp
from jax import lax
from jax.experimental import pallas as pl
from jax.experimental.pallas import tpu as pltpu
from jax.experimental.pallas import tpu_sc as plsc

_EPS = 1e-5
_B = 16384
_N = 1000000
_EMB = 64

_NC = 2
_NS = 16
_NW = _NC * _NS
_BPW = _B // _NW

_W = 8192
_RGRID = (_N + _W - 1) // _W


def _repack_body(a_ref, b_ref, out_ref):
    eye = jnp.eye(_EMB, dtype=jnp.float32)
    at = lax.dot_general(a_ref[...], eye, (((0,), (0,)), ((), ())),
                         preferred_element_type=jnp.float32)
    bt = lax.dot_general(b_ref[...], eye, (((0,), (0,)), ((), ())),
                         preferred_element_type=jnp.float32)
    c = jnp.concatenate([at, bt], axis=1).astype(jnp.bfloat16)
    out_ref[...] = pltpu.bitcast(c, jnp.uint32)


@functools.cache
def _make_repack(interpret=False):
    return pl.pallas_call(
        _repack_body,
        grid=(_RGRID,),
        in_specs=[
            pl.BlockSpec((_EMB, _W), lambda i: (0, i)),
            pl.BlockSpec((_EMB, _W), lambda i: (0, i)),
        ],
        out_specs=pl.BlockSpec((_W // 2, 128), lambda i: (i, 0)),
        out_shape=jax.ShapeDtypeStruct((_N // 2, 128), jnp.uint32),
        compiler_params=pltpu.CompilerParams(
            dimension_semantics=("parallel",),
        ),
        interpret=interpret,
    )


def _gather_body(idx_hbm, tbl_hbm, out, idx_v, buf_a, buf_b, sem_a, sem_b):
    wid = lax.axis_index("s") * _NC + lax.axis_index("c")
    base = wid * _BPW
    half = _BPW // 2
    pltpu.sync_copy(idx_hbm.at[pl.ds(base, _BPW)], idx_v)
    cp_a = pltpu.async_copy(tbl_hbm.at[idx_v.at[pl.ds(0, half)]], buf_a, sem_a)
    cp_b = pltpu.async_copy(tbl_hbm.at[idx_v.at[pl.ds(half, half)]], buf_b,
                            sem_b)
    cp_a.wait()
    pltpu.sync_copy(buf_a, out.at[pl.ds(base, half)])
    cp_b.wait()
    pltpu.sync_copy(buf_b, out.at[pl.ds(base + half, half)])


@functools.cache
def _make_gather():
    mesh = plsc.VectorSubcoreMesh(
        core_axis_name="c", subcore_axis_name="s",
        num_cores=_NC, num_subcores=_NS)
    half = _BPW // 2
    return pl.kernel(
        _gather_body,
        out_type=jax.ShapeDtypeStruct((_B, 128), jnp.uint32),
        mesh=mesh,
        scratch_types=[
            pltpu.VMEM((_BPW,), jnp.int32),
            pltpu.VMEM((half, 128), jnp.uint32),
            pltpu.VMEM((half, 128), jnp.uint32),
            pltpu.SemaphoreType.DMA,
            pltpu.SemaphoreType.DMA,
        ],
        compiler_params=pltpu.CompilerParams(use_tc_tiling_on_sc=True),
    )


_BB = 2048


def _unpack_rows(packed, parity_col):
    lo = lax.bitcast_convert_type(packed << 16, jnp.float32)
    hi = lax.bitcast_convert_type(packed & jnp.uint32(0xFFFF0000),
                                  jnp.float32)
    return jnp.where(parity_col == 1, hi, lo)


def _mlp_body(u_ref, i_ref, up_ref, ip_ref,
              w1, b1, g1, be1, w2, b2, g2, be2, wf, bf, out):
    inv_std = 1.0 / jnp.sqrt(1.0 + _EPS)
    u = _unpack_rows(u_ref[...], up_ref[...])
    i = _unpack_rows(i_ref[...], ip_ref[...])
    x = jnp.concatenate([u[:, _EMB:], i[:, _EMB:]], axis=1)
    h = lax.dot_general(x, w1[...], (((1,), (1,)), ((), ())),
                        preferred_element_type=jnp.float32)
    h = (h + b1[...]) * inv_std * g1[...] + be1[...]
    h = jnp.maximum(h, 0.0)
    h = lax.dot_general(h, w2[...], (((1,), (1,)), ((), ())),
                        preferred_element_type=jnp.float32)
    h = (h + b2[...]) * inv_std * g2[...] + be2[...]
    h = jnp.maximum(h, 0.0)
    gmf = u[:, :_EMB] * i[:, :_EMB]
    c = jnp.concatenate([gmf, h], axis=1)
    logit = jnp.sum(c * wf[...], axis=1, keepdims=True) + bf[0, 0]
    out[...] = 1.0 / (1.0 + jnp.exp(-logit))


@functools.cache
def _make_mlp(interpret=False):
    def whole(shape):
        return pl.BlockSpec(shape, lambda i: (0, 0))

    return pl.pallas_call(
        _mlp_body,
        grid=(_B // _BB,),
        in_specs=[
            pl.BlockSpec((_BB, 128), lambda i: (i, 0)),
            pl.BlockSpec((_BB, 128), lambda i: (i, 0)),
            pl.BlockSpec((_BB, 1), lambda i: (i, 0)),
            pl.BlockSpec((_BB, 1), lambda i: (i, 0)),
            whole((128, 128)),
            whole((1, 128)),
            whole((1, 128)),
            whole((1, 128)),
            whole((64, 128)),
            whole((1, 64)),
            whole((1, 64)),
            whole((1, 64)),
            whole((1, 128)),
            whole((1, 1)),
        ],
        out_specs=pl.BlockSpec((_BB, 1), lambda i: (i, 0)),
        out_shape=jax.ShapeDtypeStruct((_B, 1), jnp.float32),
        compiler_params=pltpu.CompilerParams(
            dimension_semantics=("arbitrary",),
        ),
        interpret=interpret,
    )


def kernel(user_ids, item_ids, U_mf, I_mf, U_mlp, I_mlp,
           W1, b1, g1, be1, W2, b2, g2, be2, Wf, bf):
    uid = user_ids.astype(jnp.int32)
    iid = item_ids.astype(jnp.int32)
    repack = _make_repack()
    ut = repack(U_mf.T, U_mlp.T)
    it = repack(I_mf.T, I_mlp.T)
    g = _make_gather()
    urows = g(uid // 2, ut)
    irows = g(iid // 2, it)
    return _make_mlp()(
        urows, irows,
        (uid % 2).astype(jnp.uint32).reshape(_B, 1),
        (iid % 2).astype(jnp.uint32).reshape(_B, 1),
        W1, b1.reshape(1, 128), g1.reshape(1, 128), be1.reshape(1, 128),
        W2, b2.reshape(1, 64), g2.reshape(1, 64), be2.reshape(1, 64),
        Wf, bf.reshape(1, 1),
    )


# speedup vs baseline: 2.8662x; 1.0502x over previous
import functools

import jax
import jax.numpy as jnp
from jax import lax
from jax.experimental import pallas as pl
from jax.experimental.pallas import tpu as pltpu
from jax.experimental.pallas import tpu_sc as plsc

_EPS = 1e-5
_B = 16384
_N = 1000000
_EMB = 64

_NC = 2
_NS = 16
_NW = _NC * _NS
_BPW = _B // _NW

_W = 12288
_RGRID = (_N + _W - 1) // _W


def _repack_body(a_ref, b_ref, out_ref):
    eye = jnp.eye(_EMB, dtype=jnp.float32)
    at = lax.dot_general(a_ref[...], eye, (((0,), (0,)), ((), ())),
                         preferred_element_type=jnp.float32)
    bt = lax.dot_general(b_ref[...], eye, (((0,), (0,)), ((), ())),
                         preferred_element_type=jnp.float32)
    c = jnp.concatenate([at, bt], axis=1).astype(jnp.bfloat16)
    out_ref[...] = pltpu.bitcast(c, jnp.uint32)


@functools.cache
def _make_repack(interpret=False):
    return pl.pallas_call(
        _repack_body,
        grid=(_RGRID,),
        in_specs=[
            pl.BlockSpec((_EMB, _W), lambda i: (0, i)),
            pl.BlockSpec((_EMB, _W), lambda i: (0, i)),
        ],
        out_specs=pl.BlockSpec((_W // 2, 128), lambda i: (i, 0)),
        out_shape=jax.ShapeDtypeStruct((_N // 2, 128), jnp.uint32),
        compiler_params=pltpu.CompilerParams(
            dimension_semantics=("parallel",),
        ),
        interpret=interpret,
    )


def _gather_body(idx_hbm, tbl_hbm, out, idx_v, buf_a, buf_b, sem_a, sem_b):
    wid = lax.axis_index("s") * _NC + lax.axis_index("c")
    base = wid * _BPW
    half = _BPW // 2
    pltpu.sync_copy(idx_hbm.at[pl.ds(base, _BPW)], idx_v)
    cp_a = pltpu.async_copy(tbl_hbm.at[idx_v.at[pl.ds(0, half)]], buf_a, sem_a)
    cp_b = pltpu.async_copy(tbl_hbm.at[idx_v.at[pl.ds(half, half)]], buf_b,
                            sem_b)
    cp_a.wait()
    pltpu.sync_copy(buf_a, out.at[pl.ds(base, half)])
    cp_b.wait()
    pltpu.sync_copy(buf_b, out.at[pl.ds(base + half, half)])


@functools.cache
def _make_gather():
    mesh = plsc.VectorSubcoreMesh(
        core_axis_name="c", subcore_axis_name="s",
        num_cores=_NC, num_subcores=_NS)
    half = _BPW // 2
    return pl.kernel(
        _gather_body,
        out_type=jax.ShapeDtypeStruct((_B, 128), jnp.uint32),
        mesh=mesh,
        scratch_types=[
            pltpu.VMEM((_BPW,), jnp.int32),
            pltpu.VMEM((half, 128), jnp.uint32),
            pltpu.VMEM((half, 128), jnp.uint32),
            pltpu.SemaphoreType.DMA,
            pltpu.SemaphoreType.DMA,
        ],
        compiler_params=pltpu.CompilerParams(use_tc_tiling_on_sc=True),
    )


_BB = 2048


def _unpack_rows(packed, parity_col):
    lo = lax.bitcast_convert_type(packed << 16, jnp.float32)
    hi = lax.bitcast_convert_type(packed & jnp.uint32(0xFFFF0000),
                                  jnp.float32)
    return jnp.where(parity_col == 1, hi, lo)


def _mlp_body(u_ref, i_ref, up_ref, ip_ref,
              w1, b1, g1, be1, w2, b2, g2, be2, wf, bf, out):
    inv_std = 1.0 / jnp.sqrt(1.0 + _EPS)
    u = _unpack_rows(u_ref[...], up_ref[...])
    i = _unpack_rows(i_ref[...], ip_ref[...])
    x = jnp.concatenate([u[:, _EMB:], i[:, _EMB:]], axis=1)
    h = lax.dot_general(x, w1[...], (((1,), (1,)), ((), ())),
                        preferred_element_type=jnp.float32)
    h = (h + b1[...]) * inv_std * g1[...] + be1[...]
    h = jnp.maximum(h, 0.0)
    h = lax.dot_general(h, w2[...], (((1,), (1,)), ((), ())),
                        preferred_element_type=jnp.float32)
    h = (h + b2[...]) * inv_std * g2[...] + be2[...]
    h = jnp.maximum(h, 0.0)
    gmf = u[:, :_EMB] * i[:, :_EMB]
    c = jnp.concatenate([gmf, h], axis=1)
    logit = jnp.sum(c * wf[...], axis=1, keepdims=True) + bf[0, 0]
    out[...] = 1.0 / (1.0 + jnp.exp(-logit))


@functools.cache
def _make_mlp(interpret=False):
    def whole(shape):
        return pl.BlockSpec(shape, lambda i: (0, 0))

    return pl.pallas_call(
        _mlp_body,
        grid=(_B // _BB,),
        in_specs=[
            pl.BlockSpec((_BB, 128), lambda i: (i, 0)),
            pl.BlockSpec((_BB, 128), lambda i: (i, 0)),
            pl.BlockSpec((_BB, 1), lambda i: (i, 0)),
            pl.BlockSpec((_BB, 1), lambda i: (i, 0)),
            whole((128, 128)),
            whole((1, 128)),
            whole((1, 128)),
            whole((1, 128)),
            whole((64, 128)),
            whole((1, 64)),
            whole((1, 64)),
            whole((1, 64)),
            whole((1, 128)),
            whole((1, 1)),
        ],
        out_specs=pl.BlockSpec((_BB, 1), lambda i: (i, 0)),
        out_shape=jax.ShapeDtypeStruct((_B, 1), jnp.float32),
        compiler_params=pltpu.CompilerParams(
            dimension_semantics=("arbitrary",),
        ),
        interpret=interpret,
    )


def kernel(user_ids, item_ids, U_mf, I_mf, U_mlp, I_mlp,
           W1, b1, g1, be1, W2, b2, g2, be2, Wf, bf):
    uid = user_ids.astype(jnp.int32)
    iid = item_ids.astype(jnp.int32)
    repack = _make_repack()
    ut = repack(U_mf.T, U_mlp.T)
    it = repack(I_mf.T, I_mlp.T)
    g = _make_gather()
    urows = g(uid // 2, ut)
    irows = g(iid // 2, it)
    return _make_mlp()(
        urows, irows,
        (uid % 2).astype(jnp.uint32).reshape(_B, 1),
        (iid % 2).astype(jnp.uint32).reshape(_B, 1),
        W1, b1.reshape(1, 128), g1.reshape(1, 128), be1.reshape(1, 128),
        W2, b2.reshape(1, 64), g2.reshape(1, 64), be2.reshape(1, 64),
        Wf, bf.reshape(1, 1),
    )
